# initial kernel scaffold (unmeasured)
import functools

import jax
import jax.numpy as jnp
from jax import lax
from jax.experimental import pallas as pl
from jax.experimental.pallas import tpu as pltpu

N_DEV = 8
N_TOK = 512
D = 256
H = 512
N_EXP = 32
E_LOCAL = N_EXP // N_DEV
ROWS = N_TOK // N_DEV


def kernel(x, router_W, route_idx, expert_W, shared_W):
    def body(
        x_ref,
        rw_ref,
        idx_ref,
        ew_ref,
        sw_ref,
        out_ref,
        send_buf,
        recv_buf,
        send_sems,
        recv_sems,
    ):
        my = lax.axis_index("i")

        barrier = pltpu.get_barrier_semaphore()
        for dd in range(1, N_DEV):
            pl.semaphore_signal(
                barrier,
                inc=1,
                device_id=((my + dd) % N_DEV,),
                device_id_type=pl.DeviceIdType.MESH,
            )
        pl.semaphore_wait(barrier, N_DEV - 1)

        xv = x_ref[...]
        scores = jnp.dot(xv, rw_ref[...], preferred_element_type=jnp.float32)
        smax = jnp.max(scores, axis=1, keepdims=True)
        ex = jnp.exp(scores - smax)
        probs = ex / jnp.sum(ex, axis=1, keepdims=True)

        idx = idx_ref[...]
        onehot = idx == lax.broadcasted_iota(jnp.int32, (N_TOK, N_EXP), 1)
        p_sel = jnp.sum(
            probs * onehot.astype(jnp.float32), axis=1, keepdims=True
        )

        xb = xv.astype(jnp.bfloat16)
        partial = jnp.zeros((N_TOK, H), jnp.float32)
        for k in range(E_LOCAL):
            e_id = my * E_LOCAL + k
            coeff = p_sel * (idx == e_id).astype(jnp.float32)
            y = jnp.dot(
                xb,
                ew_ref[k].astype(jnp.bfloat16),
                preferred_element_type=jnp.float32,
            )
            partial = partial + y * coeff

        pb = partial.astype(jnp.bfloat16)
        for dd in range(1, N_DEV):
            dst = (my + dd) % N_DEV
            send_buf[dd] = lax.dynamic_slice(pb, (dst * ROWS, 0), (ROWS, H))

        rdmas = []
        for dd in range(1, N_DEV):
            dst = (my + dd) % N_DEV
            rdma = pltpu.make_async_remote_copy(
                src_ref=send_buf.at[dd],
                dst_ref=recv_buf.at[dd],
                send_sem=send_sems.at[dd],
                recv_sem=recv_sems.at[dd],
                device_id=(dst,),
                device_id_type=pl.DeviceIdType.MESH,
            )
            rdma.start()
            rdmas.append(rdma)

        x_my = lax.dynamic_slice(xv, (my * ROWS, 0), (ROWS, D))
        acc = jnp.dot(
            x_my.astype(jnp.bfloat16),
            sw_ref[...].astype(jnp.bfloat16),
            preferred_element_type=jnp.float32,
        )
        acc = acc + lax.dynamic_slice(partial, (my * ROWS, 0), (ROWS, H))

        for dd in range(1, N_DEV):
            rdmas[dd - 1].wait_recv()
            acc = acc + recv_buf[dd].astype(jnp.float32)

        out_ref[...] = acc

        for dd in range(1, N_DEV):
            rdmas[dd - 1].wait_send()

        @functools.partial(pl.run_scoped, sem=pltpu.SemaphoreType.REGULAR)
        def _(sem):
            for dd in range(1, N_DEV):
                pl.semaphore_signal(
                    sem,
                    inc=1,
                    device_id=((my + dd) % N_DEV,),
                    device_id_type=pl.DeviceIdType.MESH,
                )
            pl.semaphore_wait(sem, N_DEV - 1)

    return pl.pallas_call(
        body,
        out_shape=jax.ShapeDtypeStruct((ROWS, H), jnp.float32),
        in_specs=[pl.BlockSpec(memory_space=pltpu.VMEM)] * 5,
        out_specs=pl.BlockSpec(memory_space=pltpu.VMEM),
        scratch_shapes=[
            pltpu.VMEM((N_DEV, ROWS, H), jnp.bfloat16),
            pltpu.VMEM((N_DEV, ROWS, H), jnp.bfloat16),
            pltpu.SemaphoreType.DMA((N_DEV,)),
            pltpu.SemaphoreType.DMA((N_DEV,)),
        ],
        compiler_params=pltpu.CompilerParams(collective_id=0),
    )(x, router_W, route_idx, expert_W, shared_W)


# baseline (device time: 19334 ns/iter reference)
import functools

import jax
import jax.numpy as jnp
from jax import lax
from jax.experimental import pallas as pl
from jax.experimental.pallas import tpu as pltpu

N_DEV = 8
N_TOK = 512
D = 256
H = 512
N_EXP = 32
E_LOCAL = N_EXP // N_DEV
ROWS = N_TOK // N_DEV


def kernel(x, router_W, route_idx, expert_W, shared_W):
    def body(
        x_ref,
        rw_ref,
        idx_ref,
        ew_ref,
        sw_ref,
        out_ref,
        pball,
        recv_buf,
        send_sems,
        recv_sems,
    ):
        my = lax.axis_index("i")

        barrier = pltpu.get_barrier_semaphore()
        for dd in range(1, N_DEV):
            pl.semaphore_signal(
                barrier,
                inc=1,
                device_id=((my + dd) % N_DEV,),
                device_id_type=pl.DeviceIdType.MESH,
            )
        pl.semaphore_wait(barrier, N_DEV - 1)

        xv = x_ref[...]
        scores = jnp.dot(xv, rw_ref[...], preferred_element_type=jnp.float32)
        smax = jnp.max(scores, axis=1, keepdims=True)
        ex = jnp.exp(scores - smax)
        probs = ex / jnp.sum(ex, axis=1, keepdims=True)

        idx = idx_ref[...]
        onehot = idx == lax.broadcasted_iota(jnp.int32, (N_TOK, N_EXP), 1)
        p_sel = jnp.sum(
            probs * onehot.astype(jnp.float32), axis=1, keepdims=True
        )

        xb = xv.astype(jnp.bfloat16)
        partial = jnp.zeros((N_TOK, H), jnp.float32)
        for k in range(E_LOCAL):
            e_id = my * E_LOCAL + k
            coeff = p_sel * (idx == e_id).astype(jnp.float32)
            y = jnp.dot(
                xb,
                ew_ref[k].astype(jnp.bfloat16),
                preferred_element_type=jnp.float32,
            )
            partial = partial + y * coeff

        pball[...] = partial.astype(jnp.bfloat16)

        rdmas = []
        for dd in range(1, N_DEV):
            dst = (my + dd) % N_DEV
            rdma = pltpu.make_async_remote_copy(
                src_ref=pball.at[pl.ds(dst * ROWS, ROWS), :],
                dst_ref=recv_buf.at[dd],
                send_sem=send_sems.at[dd],
                recv_sem=recv_sems.at[dd],
                device_id=(dst,),
                device_id_type=pl.DeviceIdType.MESH,
            )
            rdma.start()
            rdmas.append(rdma)

        x_my = x_ref[pl.ds(my * ROWS, ROWS), :]
        acc = jnp.dot(
            x_my.astype(jnp.bfloat16),
            sw_ref[...].astype(jnp.bfloat16),
            preferred_element_type=jnp.float32,
        )
        acc = acc + pball[pl.ds(my * ROWS, ROWS), :].astype(jnp.float32)

        for dd in range(1, N_DEV):
            rdmas[dd - 1].wait_recv()
            acc = acc + recv_buf[dd].astype(jnp.float32)

        out_ref[...] = acc

        for dd in range(1, N_DEV):
            rdmas[dd - 1].wait_send()

        @functools.partial(pl.run_scoped, sem=pltpu.SemaphoreType.REGULAR)
        def _(sem):
            for dd in range(1, N_DEV):
                pl.semaphore_signal(
                    sem,
                    inc=1,
                    device_id=((my + dd) % N_DEV,),
                    device_id_type=pl.DeviceIdType.MESH,
                )
            pl.semaphore_wait(sem, N_DEV - 1)

    return pl.pallas_call(
        body,
        out_shape=jax.ShapeDtypeStruct((ROWS, H), jnp.float32),
        in_specs=[pl.BlockSpec(memory_space=pltpu.VMEM)] * 5,
        out_specs=pl.BlockSpec(memory_space=pltpu.VMEM),
        scratch_shapes=[
            pltpu.VMEM((N_TOK, H), jnp.bfloat16),
            pltpu.VMEM((N_DEV, ROWS, H), jnp.bfloat16),
            pltpu.SemaphoreType.DMA((N_DEV,)),
            pltpu.SemaphoreType.DMA((N_DEV,)),
        ],
        compiler_params=pltpu.CompilerParams(collective_id=0),
    )(x, router_W, route_idx, expert_W, shared_W)


# device time: 15987 ns/iter; 1.2094x vs baseline; 1.2094x over previous
import jax
import jax.numpy as jnp
from jax import lax
from jax.experimental import pallas as pl
from jax.experimental.pallas import tpu as pltpu

N_DEV = 8
N_TOK = 512
D = 256
H = 512
N_EXP = 32
E_LOCAL = N_EXP // N_DEV
ROWS = N_TOK // N_DEV


def kernel(x, router_W, route_idx, expert_W, shared_W):
    def body(
        x_ref,
        rw_ref,
        idx_ref,
        ew_ref,
        sw_ref,
        out_ref,
        xb_ref,
        wcat_ref,
        coeff_ref,
        send_buf,
        recv_buf,
        send_sems,
        recv_sems,
    ):
        my = lax.axis_index("i")

        barrier = pltpu.get_barrier_semaphore()
        for dd in range(1, N_DEV):
            pl.semaphore_signal(
                barrier,
                inc=1,
                device_id=((my + dd) % N_DEV,),
                device_id_type=pl.DeviceIdType.MESH,
            )

        xv = x_ref[...]
        scores = jnp.dot(xv, rw_ref[...], preferred_element_type=jnp.float32)
        smax = jnp.max(scores, axis=1, keepdims=True)
        ex = jnp.exp(scores - smax)
        probs = ex / jnp.sum(ex, axis=1, keepdims=True)

        idx = idx_ref[...]
        onehot = idx == lax.broadcasted_iota(jnp.int32, (N_TOK, N_EXP), 1)
        p_sel = jnp.sum(
            probs * onehot.astype(jnp.float32), axis=1, keepdims=True
        )

        local_ids = my * E_LOCAL + lax.broadcasted_iota(
            jnp.int32, (N_TOK, E_LOCAL), 1
        )
        coeff_ref[...] = p_sel * (idx == local_ids).astype(jnp.float32)

        xb_ref[...] = xv.astype(jnp.bfloat16)
        for k in range(E_LOCAL):
            wcat_ref[:, k * H : (k + 1) * H] = ew_ref[k].astype(jnp.bfloat16)

        def chunk_for(dst):
            xc = xb_ref[pl.ds(dst * ROWS, ROWS), :]
            yc = jnp.dot(
                xc, wcat_ref[...], preferred_element_type=jnp.float32
            )
            cc = coeff_ref[pl.ds(dst * ROWS, ROWS), :]
            acc = yc[:, 0:H] * cc[:, 0:1]
            for k in range(1, E_LOCAL):
                acc = acc + yc[:, k * H : (k + 1) * H] * cc[:, k : k + 1]
            return acc

        pl.semaphore_wait(barrier, N_DEV - 1)

        rdmas = []
        for dd in range(1, N_DEV):
            dst = (my + dd) % N_DEV
            send_buf[dd] = chunk_for(dst).astype(jnp.bfloat16)
            rdma = pltpu.make_async_remote_copy(
                src_ref=send_buf.at[dd],
                dst_ref=recv_buf.at[dd],
                send_sem=send_sems.at[dd],
                recv_sem=recv_sems.at[dd],
                device_id=(dst,),
                device_id_type=pl.DeviceIdType.MESH,
            )
            rdma.start()
            rdmas.append(rdma)

        acc = chunk_for(my)
        acc = acc + jnp.dot(
            xb_ref[pl.ds(my * ROWS, ROWS), :],
            sw_ref[...].astype(jnp.bfloat16),
            preferred_element_type=jnp.float32,
        )

        for dd in range(1, N_DEV):
            rdmas[dd - 1].wait_recv()
            acc = acc + recv_buf[dd].astype(jnp.float32)

        out_ref[...] = acc

        for dd in range(1, N_DEV):
            rdmas[dd - 1].wait_send()

    return pl.pallas_call(
        body,
        out_shape=jax.ShapeDtypeStruct((ROWS, H), jnp.float32),
        in_specs=[pl.BlockSpec(memory_space=pltpu.VMEM)] * 5,
        out_specs=pl.BlockSpec(memory_space=pltpu.VMEM),
        scratch_shapes=[
            pltpu.VMEM((N_TOK, D), jnp.bfloat16),
            pltpu.VMEM((D, E_LOCAL * H), jnp.bfloat16),
            pltpu.VMEM((N_TOK, E_LOCAL), jnp.float32),
            pltpu.VMEM((N_DEV, ROWS, H), jnp.bfloat16),
            pltpu.VMEM((N_DEV, ROWS, H), jnp.bfloat16),
            pltpu.SemaphoreType.DMA((N_DEV,)),
            pltpu.SemaphoreType.DMA((N_DEV,)),
        ],
        compiler_params=pltpu.CompilerParams(collective_id=0),
    )(x, router_W, route_idx, expert_W, shared_W)
